# baseline (speedup 1.0000x reference)
import jax
import jax.numpy as jnp
from jax import lax
from jax.experimental import pallas as pl
from jax.experimental.pallas import tpu as pltpu

E = 256
T = 64
EB = 8
GRID = E // EB


def _body(*refs):
    n_in = 19
    ins = refs[:n_in]
    outs = refs[n_in:]
    (in_scr, in_socc, in_stap, in_gi, in_okid, in_oscal, in_omask,
     in_oslot, in_ocard, in_tmask, in_ttype, in_tscal, in_tovf,
     in_tslot, in_ispl, in_iself, in_scal6, in_steprow, in_stepcol) = ins
    (o_scr, o_socc, o_stap, o_gi, o_okid, o_oscal, o_omask,
     o_oslot, o_ocard, o_tmask, o_ttype, o_tscal, o_tovf,
     o_tslot, o_ispl, o_iself, o_scal6, o_newstep) = outs

    i = pl.program_id(0)

    three_d = (
        (o_scr, in_scr), (o_socc, in_socc), (o_stap, in_stap),
        (o_gi, in_gi), (o_okid, in_okid), (o_oscal, in_oscal),
        (o_omask, in_omask), (o_oslot, in_oslot), (o_ocard, in_ocard),
        (o_tmask, in_tmask), (o_ttype, in_ttype), (o_tscal, in_tscal),
        (o_tovf, in_tovf), (o_tslot, in_tslot), (o_ispl, in_ispl),
        (o_iself, in_iself), (o_scal6, in_scal6),
    )

    tt = lax.broadcasted_iota(jnp.int32, (EB, T, 1), 1)
    m = tt == in_stepcol[...]
    for o, x in three_d:
        o[...] = jnp.where(m, x[...], jnp.zeros((), o.dtype))

    @pl.when(i == 0)
    def _():
        o_newstep[...] = in_steprow[...] + 1


def kernel(env_indices, step_count, slot_card_rows, slot_occupied, slot_tapped,
           game_info, trace_kind_id, pending_kind_id, option_kind_ids,
           option_scalars, option_mask, option_ref_slot_idx, option_ref_card_row,
           target_mask, target_type_ids, target_scalars, target_overflow,
           target_ref_slot_idx, target_ref_is_player, target_ref_is_self,
           may_selected, old_log_probs, values, perspective_player_idx):
    B = E
    Z = slot_card_rows.shape[1]
    GID = game_info.shape[1]
    O = option_kind_ids.shape[1]
    OSD = option_scalars.shape[2]
    TG = target_mask.shape[2]
    TSD = target_scalars.shape[3]

    def pack_bool(x):
        return lax.bitcast_convert_type(x.astype(jnp.uint8), jnp.uint32)

    ispl = pack_bool(target_ref_is_player)
    iself = pack_bool(target_ref_is_self)

    as_f32 = lambda v: lax.bitcast_convert_type(v, jnp.float32)
    scal6 = jnp.stack(
        [as_f32(trace_kind_id), as_f32(pending_kind_id), may_selected,
         old_log_probs, values, as_f32(perspective_player_idx)], axis=1)

    flat_ins = [
        slot_card_rows, slot_occupied, slot_tapped, game_info,
        option_kind_ids, option_scalars.reshape(B, O * OSD), option_mask,
        option_ref_slot_idx, option_ref_card_row,
        target_mask.reshape(B, O * TG), target_type_ids.reshape(B, O * TG),
        target_scalars.reshape(B, O * TG * TSD), target_overflow,
        target_ref_slot_idx.reshape(B, O * TG), ispl, iself, scal6,
    ]
    flat_ins = [x.reshape(B, 1, x.shape[-1]) for x in flat_ins]
    flat_ins.append(step_count.reshape(1, E))
    flat_ins.append(step_count.reshape(E, 1, 1))

    rests = [Z, Z, Z, GID, O, O * OSD, O, O, O, O * TG, O * TG,
             O * TG * TSD, O, O * TG, O, O, 6]
    dtypes = [jnp.int32, jnp.float32, jnp.float32, jnp.float32,
              jnp.int32, jnp.float32, jnp.float32, jnp.int32, jnp.int32,
              jnp.float32, jnp.int32, jnp.float32, jnp.float32, jnp.int32,
              jnp.uint32, jnp.uint32, jnp.float32]

    out_shapes = [jax.ShapeDtypeStruct((E, T, r), d) for r, d in zip(rests, dtypes)]
    out_shapes.append(jax.ShapeDtypeStruct((1, E), jnp.int32))

    in_specs = [pl.BlockSpec((EB, 1, r), lambda i: (i, 0, 0)) for r in rests]
    in_specs.append(pl.BlockSpec((1, E), lambda i: (0, 0)))
    in_specs.append(pl.BlockSpec((EB, 1, 1), lambda i: (i, 0, 0)))
    out_specs = [pl.BlockSpec((EB, T, r), lambda i: (i, 0, 0)) for r in rests]
    out_specs.append(pl.BlockSpec((1, E), lambda i: (0, 0)))

    outs = pl.pallas_call(
        _body,
        grid=(GRID,),
        in_specs=in_specs,
        out_specs=out_specs,
        out_shape=out_shapes,
        compiler_params=pltpu.CompilerParams(
            dimension_semantics=("arbitrary",),
        ),
    )(*flat_ins)

    (b_scr, b_socc, b_stap, b_gi, b_okid, b_oscal, b_omask, b_oslot,
     b_ocard, b_tmask, b_ttype, b_tscal, b_tovf, b_tslot, b_ispl,
     b_iself, b_scal6, b_newstep) = outs

    def unpack_bool(x):
        return lax.bitcast_convert_type(x, jnp.uint8).astype(jnp.bool_)

    as_i32 = lambda v: lax.bitcast_convert_type(v, jnp.int32)

    return (
        b_scr, b_socc, b_stap, b_gi,
        as_i32(b_scal6[:, :, 0]), as_i32(b_scal6[:, :, 1]),
        b_okid, b_oscal.reshape(E, T, O, OSD), b_omask, b_oslot, b_ocard,
        b_tmask.reshape(E, T, O, TG), b_ttype.reshape(E, T, O, TG),
        b_tscal.reshape(E, T, O, TG, TSD), b_tovf,
        b_tslot.reshape(E, T, O, TG), unpack_bool(b_ispl), unpack_bool(b_iself),
        b_scal6[:, :, 2], b_scal6[:, :, 3], b_scal6[:, :, 4],
        as_i32(b_scal6[:, :, 5]),
        b_newstep.reshape(E),
    )


# speedup vs baseline: 6.4485x; 6.4485x over previous
import jax
import jax.numpy as jnp
from jax import lax
from jax.experimental import pallas as pl
from jax.experimental.pallas import tpu as pltpu

E = 256
T = 64
TB = 8
GRID = T // TB


def _body(*refs):
    n_in = 26
    ins = refs[:n_in]
    outs = refs[n_in:]
    (in_scr, in_socc, in_stap, in_gi, in_okid, in_omask, in_oslot,
     in_ocard, in_tovf, in_ispl, in_iself, in_oscal, in_tmask, in_ttype,
     in_tslot, in_tscal, in_trace, in_pend, in_may, in_olp, in_val,
     in_persp) = ins[:22]
    (o_scr, o_socc, o_stap, o_gi, o_okid, o_omask, o_oslot, o_ocard,
     o_tovf, o_ispl, o_iself, o_oscal, o_tmask, o_ttype, o_tslot,
     o_tscal, o_trace, o_pend, o_may, o_olp, o_val, o_persp,
     o_newstep) = outs

    st2 = ins[22]
    base = pl.program_id(0) * TB

    steps2 = st2[...]
    t2 = lax.broadcasted_iota(jnp.int32, (TB, E), 0) + base
    m2 = t2 == steps2
    t3 = lax.broadcasted_iota(jnp.int32, (TB, 1, E), 0) + base
    m3 = t3 == ins[23][...]
    t4 = lax.broadcasted_iota(jnp.int32, (TB, 1, 1, E), 0) + base
    m4 = t4 == ins[24][...]
    t5 = lax.broadcasted_iota(jnp.int32, (TB, 1, 1, 1, E), 0) + base
    m5 = t5 == ins[25][...]

    r3 = ((o_scr, in_scr), (o_socc, in_socc), (o_stap, in_stap),
          (o_gi, in_gi), (o_okid, in_okid), (o_omask, in_omask),
          (o_oslot, in_oslot), (o_ocard, in_ocard), (o_tovf, in_tovf),
          (o_ispl, in_ispl), (o_iself, in_iself))
    for o, x in r3:
        o[...] = jnp.where(m3, x[...], jnp.zeros((), o.dtype))

    r4 = ((o_oscal, in_oscal), (o_tmask, in_tmask), (o_ttype, in_ttype),
          (o_tslot, in_tslot))
    for o, x in r4:
        o[...] = jnp.where(m4, x[...], jnp.zeros((), o.dtype))

    o_tscal[...] = jnp.where(m5, in_tscal[...], jnp.zeros((), o_tscal.dtype))

    r2 = ((o_trace, in_trace), (o_pend, in_pend), (o_may, in_may),
          (o_olp, in_olp), (o_val, in_val), (o_persp, in_persp))
    for o, x in r2:
        o[...] = jnp.where(m2, x[...], jnp.zeros((), o.dtype))

    @pl.when(pl.program_id(0) == 0)
    def _():
        o_newstep[...] = steps2 + 1


def kernel(env_indices, step_count, slot_card_rows, slot_occupied, slot_tapped,
           game_info, trace_kind_id, pending_kind_id, option_kind_ids,
           option_scalars, option_mask, option_ref_slot_idx, option_ref_card_row,
           target_mask, target_type_ids, target_scalars, target_overflow,
           target_ref_slot_idx, target_ref_is_player, target_ref_is_self,
           may_selected, old_log_probs, values, perspective_player_idx):
    B = E
    Z = slot_card_rows.shape[1]
    GID = game_info.shape[1]
    O = option_kind_ids.shape[1]
    OSD = option_scalars.shape[2]
    TG = target_mask.shape[2]
    TSD = target_scalars.shape[3]

    def pack_bool(x):
        return lax.bitcast_convert_type(x.astype(jnp.uint8), jnp.uint32)

    t2d = lambda x: x.T.reshape(1, x.shape[1], B)
    r3_ins = [t2d(slot_card_rows), t2d(slot_occupied), t2d(slot_tapped),
              t2d(game_info), t2d(option_kind_ids), t2d(option_mask),
              t2d(option_ref_slot_idx), t2d(option_ref_card_row),
              t2d(target_overflow), t2d(pack_bool(target_ref_is_player)),
              t2d(pack_bool(target_ref_is_self))]
    r4_ins = [option_scalars.transpose(1, 2, 0).reshape(1, O, OSD, E),
              target_mask.transpose(1, 2, 0).reshape(1, O, TG, E),
              target_type_ids.transpose(1, 2, 0).reshape(1, O, TG, E),
              target_ref_slot_idx.transpose(1, 2, 0).reshape(1, O, TG, E)]
    r5_in = target_scalars.transpose(1, 2, 3, 0).reshape(1, O, TG, TSD, E)
    r2_ins = [x.reshape(1, E) for x in (trace_kind_id, pending_kind_id,
                                        may_selected, old_log_probs, values,
                                        perspective_player_idx)]
    step_ins = [step_count.reshape((1,) * k + (E,)) for k in (1, 2, 3, 4)]

    flat_ins = r3_ins + [r4_ins[0]] + r4_ins[1:] + [r5_in] + r2_ins + step_ins
    flat_ins = (r3_ins + r4_ins + [r5_in] + r2_ins + step_ins)

    r3_shapes = [(T, Z, E), (T, Z, E), (T, Z, E), (T, GID, E), (T, O, E),
                 (T, O, E), (T, O, E), (T, O, E), (T, O, E), (T, O, E),
                 (T, O, E)]
    r3_dtypes = [jnp.int32, jnp.float32, jnp.float32, jnp.float32, jnp.int32,
                 jnp.float32, jnp.int32, jnp.int32, jnp.float32, jnp.uint32,
                 jnp.uint32]
    r4_shapes = [(T, O, OSD, E), (T, O, TG, E), (T, O, TG, E), (T, O, TG, E)]
    r4_dtypes = [jnp.float32, jnp.float32, jnp.int32, jnp.int32]
    r2_dtypes = [jnp.int32, jnp.int32, jnp.float32, jnp.float32, jnp.float32,
                 jnp.int32]

    out_shapes = (
        [jax.ShapeDtypeStruct(s, d) for s, d in zip(r3_shapes, r3_dtypes)]
        + [jax.ShapeDtypeStruct(s, d) for s, d in zip(r4_shapes, r4_dtypes)]
        + [jax.ShapeDtypeStruct((T, O, TG, TSD, E), jnp.float32)]
        + [jax.ShapeDtypeStruct((T, E), d) for d in r2_dtypes]
        + [jax.ShapeDtypeStruct((1, E), jnp.int32)]
    )

    c0 = lambda r: (lambda i, _r=r: (0,) * _r)
    in_specs = (
        [pl.BlockSpec((1,) + s[1:], c0(3)) for s in r3_shapes]
        + [pl.BlockSpec((1,) + s[1:], c0(4)) for s in r4_shapes]
        + [pl.BlockSpec((1, O, TG, TSD, E), c0(5))]
        + [pl.BlockSpec((1, E), c0(2))] * 6
        + [pl.BlockSpec((1,) * k + (E,), c0(k + 1)) for k in (1, 2, 3, 4)]
    )
    lead = lambda r: (lambda i, _r=r: (i,) + (0,) * (_r - 1))
    out_specs = (
        [pl.BlockSpec((TB,) + s[1:], lead(3)) for s in r3_shapes]
        + [pl.BlockSpec((TB,) + s[1:], lead(4)) for s in r4_shapes]
        + [pl.BlockSpec((TB, O, TG, TSD, E), lead(5))]
        + [pl.BlockSpec((TB, E), lead(2))] * 6
        + [pl.BlockSpec((1, E), c0(2))]
    )

    outs = pl.pallas_call(
        _body,
        grid=(GRID,),
        in_specs=in_specs,
        out_specs=out_specs,
        out_shape=out_shapes,
        compiler_params=pltpu.CompilerParams(
            dimension_semantics=("arbitrary",),
        ),
    )(*flat_ins)

    (b_scr, b_socc, b_stap, b_gi, b_okid, b_omask, b_oslot, b_ocard,
     b_tovf, b_ispl, b_iself, b_oscal, b_tmask, b_ttype, b_tslot,
     b_tscal, b_trace, b_pend, b_may, b_olp, b_val, b_persp,
     b_newstep) = outs

    tr3 = lambda x: jnp.transpose(x, (2, 0, 1))
    tr4 = lambda x: jnp.transpose(x, (3, 0, 1, 2))
    tr5 = lambda x: jnp.transpose(x, (4, 0, 1, 2, 3))

    def unpack_bool(x):
        return lax.bitcast_convert_type(tr3(x), jnp.uint8).astype(jnp.bool_)

    return (
        tr3(b_scr), tr3(b_socc), tr3(b_stap), tr3(b_gi),
        b_trace.T, b_pend.T,
        tr3(b_okid), tr4(b_oscal), tr3(b_omask), tr3(b_oslot), tr3(b_ocard),
        tr4(b_tmask), tr4(b_ttype), tr5(b_tscal), tr3(b_tovf),
        tr4(b_tslot), unpack_bool(b_ispl), unpack_bool(b_iself),
        b_may.T, b_olp.T, b_val.T, b_persp.T,
        b_newstep.reshape(E),
    )


# speedup vs baseline: 7.2209x; 1.1198x over previous
import jax
import jax.numpy as jnp
from jax import lax
from jax.experimental import pallas as pl
from jax.experimental.pallas import tpu as pltpu

E = 256
T = 64
TB = 8
GRID = T // TB


def _body(*refs):
    n_in = 24
    ins = refs[:n_in]
    outs = refs[n_in:]
    (in_scr, in_socc, in_stap, in_gi, in_okid, in_omask, in_oslot,
     in_ocard, in_tovf, in_oscal, in_tmask, in_ttype,
     in_tslot, in_tscal, in_trace, in_pend, in_may, in_olp, in_val,
     in_persp) = ins[:20]
    (o_scr, o_socc, o_stap, o_gi, o_okid, o_omask, o_oslot, o_ocard,
     o_tovf, o_oscal, o_tmask, o_ttype, o_tslot,
     o_tscal, o_trace, o_pend, o_may, o_olp, o_val, o_persp,
     o_newstep) = outs

    st2 = ins[20]
    base = pl.program_id(0) * TB

    steps2 = st2[...]
    t2 = lax.broadcasted_iota(jnp.int32, (TB, E), 0) + base
    m2 = t2 == steps2
    t3 = lax.broadcasted_iota(jnp.int32, (TB, 1, E), 0) + base
    m3 = t3 == ins[21][...]
    t4 = lax.broadcasted_iota(jnp.int32, (TB, 1, 1, E), 0) + base
    m4 = t4 == ins[22][...]
    t5 = lax.broadcasted_iota(jnp.int32, (TB, 1, 1, 1, E), 0) + base
    m5 = t5 == ins[23][...]

    r3 = ((o_scr, in_scr), (o_socc, in_socc), (o_stap, in_stap),
          (o_gi, in_gi), (o_okid, in_okid), (o_omask, in_omask),
          (o_oslot, in_oslot), (o_ocard, in_ocard), (o_tovf, in_tovf))
    for o, x in r3:
        o[...] = jnp.where(m3, x[...], jnp.zeros((), o.dtype))

    r4 = ((o_oscal, in_oscal), (o_tmask, in_tmask), (o_ttype, in_ttype),
          (o_tslot, in_tslot))
    for o, x in r4:
        o[...] = jnp.where(m4, x[...], jnp.zeros((), o.dtype))

    o_tscal[...] = jnp.where(m5, in_tscal[...], jnp.zeros((), o_tscal.dtype))

    r2 = ((o_trace, in_trace), (o_pend, in_pend), (o_may, in_may),
          (o_olp, in_olp), (o_val, in_val), (o_persp, in_persp))
    for o, x in r2:
        o[...] = jnp.where(m2, x[...], jnp.zeros((), o.dtype))

    @pl.when(pl.program_id(0) == 0)
    def _():
        o_newstep[...] = steps2 + 1


def kernel(env_indices, step_count, slot_card_rows, slot_occupied, slot_tapped,
           game_info, trace_kind_id, pending_kind_id, option_kind_ids,
           option_scalars, option_mask, option_ref_slot_idx, option_ref_card_row,
           target_mask, target_type_ids, target_scalars, target_overflow,
           target_ref_slot_idx, target_ref_is_player, target_ref_is_self,
           may_selected, old_log_probs, values, perspective_player_idx):
    B = E
    Z = slot_card_rows.shape[1]
    GID = game_info.shape[1]
    O = option_kind_ids.shape[1]
    OSD = option_scalars.shape[2]
    TG = target_mask.shape[2]
    TSD = target_scalars.shape[3]

    t2d = lambda x: x.T.reshape(1, x.shape[1], B)
    r3_ins = [t2d(slot_card_rows), t2d(slot_occupied), t2d(slot_tapped),
              t2d(game_info), t2d(option_kind_ids), t2d(option_mask),
              t2d(option_ref_slot_idx), t2d(option_ref_card_row),
              t2d(target_overflow)]
    r4_ins = [option_scalars.transpose(1, 2, 0).reshape(1, O, OSD, E),
              target_mask.transpose(1, 2, 0).reshape(1, O, TG, E),
              target_type_ids.transpose(1, 2, 0).reshape(1, O, TG, E),
              target_ref_slot_idx.transpose(1, 2, 0).reshape(1, O, TG, E)]
    r5_in = target_scalars.transpose(1, 2, 3, 0).reshape(1, O, TG, TSD, E)
    r2_ins = [x.reshape(1, E) for x in (trace_kind_id, pending_kind_id,
                                        may_selected, old_log_probs, values,
                                        perspective_player_idx)]
    step_ins = [step_count.reshape((1,) * k + (E,)) for k in (1, 2, 3, 4)]

    flat_ins = r3_ins + [r4_ins[0]] + r4_ins[1:] + [r5_in] + r2_ins + step_ins
    flat_ins = (r3_ins + r4_ins + [r5_in] + r2_ins + step_ins)

    r3_shapes = [(T, Z, E), (T, Z, E), (T, Z, E), (T, GID, E), (T, O, E),
                 (T, O, E), (T, O, E), (T, O, E), (T, O, E)]
    r3_dtypes = [jnp.int32, jnp.float32, jnp.float32, jnp.float32, jnp.int32,
                 jnp.float32, jnp.int32, jnp.int32, jnp.float32]
    r4_shapes = [(T, O, OSD, E), (T, O, TG, E), (T, O, TG, E), (T, O, TG, E)]
    r4_dtypes = [jnp.float32, jnp.float32, jnp.int32, jnp.int32]
    r2_dtypes = [jnp.int32, jnp.int32, jnp.float32, jnp.float32, jnp.float32,
                 jnp.int32]

    out_shapes = (
        [jax.ShapeDtypeStruct(s, d) for s, d in zip(r3_shapes, r3_dtypes)]
        + [jax.ShapeDtypeStruct(s, d) for s, d in zip(r4_shapes, r4_dtypes)]
        + [jax.ShapeDtypeStruct((T, O, TG, TSD, E), jnp.float32)]
        + [jax.ShapeDtypeStruct((T, E), d) for d in r2_dtypes]
        + [jax.ShapeDtypeStruct((1, E), jnp.int32)]
    )

    c0 = lambda r: (lambda i, _r=r: (0,) * _r)
    in_specs = (
        [pl.BlockSpec((1,) + s[1:], c0(3)) for s in r3_shapes]
        + [pl.BlockSpec((1,) + s[1:], c0(4)) for s in r4_shapes]
        + [pl.BlockSpec((1, O, TG, TSD, E), c0(5))]
        + [pl.BlockSpec((1, E), c0(2))] * 6
        + [pl.BlockSpec((1,) * k + (E,), c0(k + 1)) for k in (1, 2, 3, 4)]
    )
    lead = lambda r: (lambda i, _r=r: (i,) + (0,) * (_r - 1))
    out_specs = (
        [pl.BlockSpec((TB,) + s[1:], lead(3)) for s in r3_shapes]
        + [pl.BlockSpec((TB,) + s[1:], lead(4)) for s in r4_shapes]
        + [pl.BlockSpec((TB, O, TG, TSD, E), lead(5))]
        + [pl.BlockSpec((TB, E), lead(2))] * 6
        + [pl.BlockSpec((1, E), c0(2))]
    )

    outs = pl.pallas_call(
        _body,
        grid=(GRID,),
        in_specs=in_specs,
        out_specs=out_specs,
        out_shape=out_shapes,
        compiler_params=pltpu.CompilerParams(
            dimension_semantics=("arbitrary",),
        ),
    )(*flat_ins)

    (b_scr, b_socc, b_stap, b_gi, b_okid, b_omask, b_oslot, b_ocard,
     b_tovf, b_oscal, b_tmask, b_ttype, b_tslot,
     b_tscal, b_trace, b_pend, b_may, b_olp, b_val, b_persp,
     b_newstep) = outs

    tr3 = lambda x: jnp.transpose(x, (2, 0, 1))
    tr4 = lambda x: jnp.transpose(x, (3, 0, 1, 2))
    tr5 = lambda x: jnp.transpose(x, (4, 0, 1, 2, 3))

    onehot = step_count[:, None] == jnp.arange(T, dtype=jnp.int32)[None, :]
    mb = onehot[:, :, None, None]
    b_ispl = mb & target_ref_is_player[:, None, :, :]
    b_iself = mb & target_ref_is_self[:, None, :, :]

    return (
        tr3(b_scr), tr3(b_socc), tr3(b_stap), tr3(b_gi),
        b_trace.T, b_pend.T,
        tr3(b_okid), tr4(b_oscal), tr3(b_omask), tr3(b_oslot), tr3(b_ocard),
        tr4(b_tmask), tr4(b_ttype), tr5(b_tscal), tr3(b_tovf),
        tr4(b_tslot), b_ispl, b_iself,
        b_may.T, b_olp.T, b_val.T, b_persp.T,
        b_newstep.reshape(E),
    )
